# baseline (speedup 1.0000x reference)
import functools

import jax
import jax.numpy as jnp
from jax import lax
from jax.experimental import pallas as pl
from jax.experimental.pallas import tpu as pltpu
from jax.experimental.pallas import tpu_sc as plsc

NENTITY = 1000000
NRELATION = 1000
DIM = 64
BATCH = 1024
NEG = 256

L = 16
NC = 2
NS = 16
NW = NC * NS
BPW = BATCH // NW
NEG_HALF = NEG // 2

_mesh = plsc.VectorSubcoreMesh(core_axis_name="c", subcore_axis_name="s")


@functools.partial(
    pl.kernel,
    mesh=_mesh,
    compiler_params=pltpu.CompilerParams(use_tc_tiling_on_sc=False),
    out_type=jax.ShapeDtypeStruct((BATCH, NEG), jnp.float32),
    scratch_types=[
        pltpu.VMEM((BPW,), jnp.int32),
        pltpu.VMEM((BPW,), jnp.int32),
        pltpu.VMEM((BPW, 2, NEG_HALF), jnp.int32),
        pltpu.VMEM((BPW, DIM), jnp.float32),
        pltpu.VMEM((BPW, DIM), jnp.float32),
        pltpu.VMEM((NEG, DIM), jnp.float32),
        pltpu.VMEM((NEG, DIM), jnp.float32),
        pltpu.VMEM((BPW, NEG), jnp.float32),
        pltpu.SemaphoreType.DMA,
        pltpu.SemaphoreType.DMA,
    ],
)
def _kge_sc(hidx_hbm, ridx_hbm, neg_hbm, ent_hbm, rel_hbm, out_hbm,
            hidx_v, ridx_v, nidx_v, hrows_v, rrows_v, tail0_v, tail1_v,
            score_v, sem0, sem1):
    wid = lax.axis_index("s") * NC + lax.axis_index("c")
    base = wid * BPW

    pltpu.sync_copy(hidx_hbm.at[pl.ds(base, BPW)], hidx_v)
    pltpu.sync_copy(ridx_hbm.at[pl.ds(base, BPW)], ridx_v)
    pltpu.sync_copy(neg_hbm.at[pl.ds(base, BPW)], nidx_v)
    pltpu.async_copy(ent_hbm.at[hidx_v], hrows_v, sem0).wait()
    pltpu.async_copy(rel_hbm.at[ridx_v], rrows_v, sem0).wait()

    def hr_body(b, carry):
        for k in range(DIM // L):
            sl = pl.ds(k * L, L)
            rrows_v[b, sl] = hrows_v[b, sl] * rrows_v[b, sl]
        return carry

    lax.fori_loop(0, BPW, hr_body, 0)

    def tail_copies(b, buf, sem):
        return (
            pltpu.make_async_copy(ent_hbm.at[nidx_v.at[b, 0]],
                                  buf.at[pl.ds(0, NEG_HALF)], sem),
            pltpu.make_async_copy(ent_hbm.at[nidx_v.at[b, 1]],
                                  buf.at[pl.ds(NEG_HALF, NEG_HALF)], sem),
        )

    def start_tails(b, buf, sem):
        for cp in tail_copies(b, buf, sem):
            cp.start()

    def wait_tails(b, buf, sem):
        for cp in tail_copies(b, buf, sem):
            cp.wait()

    lanes = lax.iota(jnp.int32, L)
    dnums = lax.GatherDimensionNumbers(
        offset_dims=(), collapsed_slice_dims=(0,), start_index_map=(0,))
    perm_idx = {k: lanes ^ k for k in (1, 2, 4, 8)}
    merge_mask = {k: (lanes & k) != 0 for k in (1, 2, 4, 8)}

    def merge(a, b, k):
        m = merge_mask[k]
        t = jnp.where(m, b, a)
        u = jnp.where(m, a, b)
        return t + lax.gather(u, perm_idx[k][:, None], dnums, (1,),
                              mode=lax.GatherScatterMode.PROMISE_IN_BOUNDS)

    def compute_row(b, tail_v):
        hr0 = rrows_v[b, pl.ds(0 * L, L)]
        hr1 = rrows_v[b, pl.ds(1 * L, L)]
        hr2 = rrows_v[b, pl.ds(2 * L, L)]
        hr3 = rrows_v[b, pl.ds(3 * L, L)]

        def grp_body(g, gcarry):
            n0 = g * L
            s = []
            for j in range(L):
                n = n0 + j
                s.append(tail_v[n, pl.ds(0 * L, L)] * hr0
                         + tail_v[n, pl.ds(1 * L, L)] * hr1
                         + tail_v[n, pl.ds(2 * L, L)] * hr2
                         + tail_v[n, pl.ds(3 * L, L)] * hr3)
            t = [merge(s[2 * i], s[2 * i + 1], 1) for i in range(8)]
            u = [merge(t[2 * i], t[2 * i + 1], 2) for i in range(4)]
            v = [merge(u[2 * i], u[2 * i + 1], 4) for i in range(2)]
            score_v[b, pl.ds(n0, L)] = merge(v[0], v[1], 8)
            return gcarry

        lax.fori_loop(0, NEG // L, grp_body, 0)

    start_tails(0, tail0_v, sem0)

    def pair_body(i, carry):
        b0 = 2 * i
        b1 = b0 + 1
        start_tails(b1, tail1_v, sem1)
        wait_tails(b0, tail0_v, sem0)
        compute_row(b0, tail0_v)

        @pl.when(i < BPW // 2 - 1)
        def _():
            start_tails(b0 + 2, tail0_v, sem0)

        wait_tails(b1, tail1_v, sem1)
        compute_row(b1, tail1_v)
        return carry

    lax.fori_loop(0, BPW // 2, pair_body, 0)
    pltpu.sync_copy(score_v, out_hbm.at[pl.ds(base, BPW)])


def kernel(pos_part, neg_part, entity_embedding, relation_embedding):
    hidx = pos_part[:, 0].astype(jnp.int32)
    ridx = pos_part[:, 1].astype(jnp.int32)
    neg3 = neg_part.astype(jnp.int32).reshape(BATCH, 2, NEG_HALF)
    return _kge_sc(hidx, ridx, neg3, entity_embedding, relation_embedding)
